# baseline (speedup 1.0000x reference)
import functools

import jax
import jax.numpy as jnp
from jax.experimental import pallas as pl

TM = 256
KWIN = 2736
INT_MAX = 2**31 - 1


def _wnorm_kernel(w_ref, out_ref):
    w = w_ref[...]
    out_ref[...] = jnp.sum(w * w, axis=1, keepdims=True)


def _xnorm_kernel(x_ref, out_ref):
    x = x_ref[0]
    out_ref[...] = jnp.sum(x * x, axis=0, keepdims=True).reshape(1, 1, TM)


def _vq_kernel(x_ref, xn_ref, w_ref, wn_ref, out_ref):
    x = x_ref[0]
    xn = xn_ref[0]
    k_total = w_ref.shape[0]
    r_val = jnp.full((1, TM), jnp.inf, dtype=jnp.float32)
    r_idx = jnp.zeros((1, TM), dtype=jnp.int32)
    k0 = 0
    while k0 < k_total:
        kt = min(KWIN, k_total - k0)
        w_tile = w_ref[pl.ds(k0, kt), :]
        wn_tile = wn_ref[pl.ds(k0, kt), :]
        s = jax.lax.dot_general(
            w_tile, x, (((1,), (0,)), ((), ())),
            preferred_element_type=jnp.float32)
        d = (xn + wn_tile) - 2.0 * s
        m = jnp.min(d, axis=0, keepdims=True)
        iota = jax.lax.broadcasted_iota(jnp.int32, (kt, TM), 0) + k0
        idx = jnp.min(jnp.where(d == m, iota, INT_MAX), axis=0, keepdims=True)
        take = m < r_val
        r_idx = jnp.where(take, idx, r_idx)
        r_val = jnp.where(take, m, r_val)
        r_val = r_val.astype(jnp.bfloat16).astype(jnp.float32)
        k0 += kt
    out_ref[...] = r_idx.reshape(1, 1, TM)


@functools.partial(jax.jit, static_argnames=("interpret",))
def kernel(inputs, weight, interpret=False):
    B, C, T = inputs.shape
    K = weight.shape[0]
    tt = T // TM

    wnorm = pl.pallas_call(
        _wnorm_kernel,
        out_shape=jax.ShapeDtypeStruct((K, 1), jnp.float32),
        interpret=interpret,
    )(weight)

    xnorm = pl.pallas_call(
        _xnorm_kernel,
        grid=(B * tt,),
        in_specs=[pl.BlockSpec((1, C, TM), lambda n, tt=tt: (n // tt, 0, n % tt))],
        out_specs=pl.BlockSpec((1, 1, TM), lambda n: (n, 0, 0)),
        out_shape=jax.ShapeDtypeStruct((B * tt, 1, TM), jnp.float32),
        interpret=interpret,
    )(inputs)

    x_bf = inputs.astype(jnp.bfloat16)
    w_bf = weight.astype(jnp.bfloat16)

    out = pl.pallas_call(
        _vq_kernel,
        grid=(B * tt,),
        in_specs=[
            pl.BlockSpec((1, C, TM), lambda n, tt=tt: (n // tt, 0, n % tt)),
            pl.BlockSpec((1, 1, TM), lambda n: (n, 0, 0)),
            pl.BlockSpec((K, C), lambda n: (0, 0)),
            pl.BlockSpec((K, 1), lambda n: (0, 0)),
        ],
        out_specs=pl.BlockSpec((1, 1, TM), lambda n: (n, 0, 0)),
        out_shape=jax.ShapeDtypeStruct((B * tt, 1, TM), jnp.int32),
        interpret=interpret,
    )(x_bf, xnorm, w_bf, wnorm)

    return out.reshape(B, T)


# speedup vs baseline: 1.2423x; 1.2423x over previous
import functools

import jax
import jax.numpy as jnp
from jax.experimental import pallas as pl

TM = 512
KWIN = 2736
INT_MAX = 2**31 - 1


def _wnorm_kernel(w_ref, out_ref):
    w = w_ref[...]
    out_ref[...] = jnp.sum(w * w, axis=1, keepdims=True)


def _xnorm_kernel(x_ref, out_ref):
    x = x_ref[0]
    out_ref[...] = jnp.sum(x * x, axis=0, keepdims=True).reshape(1, 1, TM)


def _vq_kernel(x_ref, xn_ref, w_ref, wn_ref, out_ref):
    x = x_ref[0]
    xn = xn_ref[0]
    k_total = w_ref.shape[0]
    r_val = jnp.full((1, TM), jnp.inf, dtype=jnp.float32)
    r_idx = jnp.zeros((1, TM), dtype=jnp.int32)
    k0 = 0
    while k0 < k_total:
        kt = min(KWIN, k_total - k0)
        w_tile = w_ref[pl.ds(k0, kt), :]
        wn_tile = wn_ref[pl.ds(k0, kt), :]
        s2 = jax.lax.dot_general(
            w_tile, x, (((1,), (0,)), ((), ())),
            preferred_element_type=jnp.float32)
        d = (xn + wn_tile) - s2
        m = jnp.min(d, axis=0, keepdims=True)
        iota = jax.lax.broadcasted_iota(jnp.int32, (kt, TM), 0) + k0
        idx = jnp.min(jnp.where(d == m, iota, INT_MAX), axis=0, keepdims=True)
        take = m < r_val
        r_idx = jnp.where(take, idx, r_idx)
        r_val = jnp.where(take, m, r_val)
        r_val = r_val.astype(jnp.bfloat16).astype(jnp.float32)
        k0 += kt
    out_ref[...] = r_idx.reshape(1, 1, TM)


@functools.partial(jax.jit, static_argnames=("interpret",))
def kernel(inputs, weight, interpret=False):
    B, C, T = inputs.shape
    K = weight.shape[0]
    tt = T // TM

    wnorm = pl.pallas_call(
        _wnorm_kernel,
        out_shape=jax.ShapeDtypeStruct((K, 1), jnp.float32),
        interpret=interpret,
    )(weight)

    xnorm = pl.pallas_call(
        _xnorm_kernel,
        grid=(B * tt,),
        in_specs=[pl.BlockSpec((1, C, TM), lambda n, tt=tt: (n // tt, 0, n % tt))],
        out_specs=pl.BlockSpec((1, 1, TM), lambda n: (n, 0, 0)),
        out_shape=jax.ShapeDtypeStruct((B * tt, 1, TM), jnp.float32),
        interpret=interpret,
    )(inputs)

    x_bf = (inputs.astype(jnp.bfloat16) * 2)
    w_bf = weight.astype(jnp.bfloat16)

    out = pl.pallas_call(
        _vq_kernel,
        grid=(B * tt,),
        in_specs=[
            pl.BlockSpec((1, C, TM), lambda n, tt=tt: (n // tt, 0, n % tt)),
            pl.BlockSpec((1, 1, TM), lambda n: (n, 0, 0)),
            pl.BlockSpec((K, C), lambda n: (0, 0)),
            pl.BlockSpec((K, 1), lambda n: (0, 0)),
        ],
        out_specs=pl.BlockSpec((1, 1, TM), lambda n: (n, 0, 0)),
        out_shape=jax.ShapeDtypeStruct((B * tt, 1, TM), jnp.int32),
        interpret=interpret,
    )(x_bf, xnorm, w_bf, wnorm)

    return out.reshape(B, T)


# speedup vs baseline: 1.2798x; 1.0302x over previous
import functools

import jax
import jax.numpy as jnp
from jax.experimental import pallas as pl

TM = 512
KWIN = 2736
INT_MAX = 2**31 - 1


def _wnorm_kernel(w_ref, out_ref):
    w = w_ref[...]
    out_ref[...] = jnp.sum(w * w, axis=1, keepdims=True)


def _xnorm_kernel(x_ref, out_ref):
    x = x_ref[0]
    out_ref[...] = jnp.sum(x * x, axis=0, keepdims=True).reshape(1, 1, TM)


CHUNK = 304


def _vq_kernel(x_ref, xn_ref, w_ref, wn_ref, out_ref):
    x = x_ref[0]
    xn = xn_ref[0]
    k_total = w_ref.shape[0]
    r_val = jnp.full((1, TM), jnp.inf, dtype=jnp.float32)
    r_idx = jnp.zeros((1, TM), dtype=jnp.int32)
    k0 = 0
    while k0 < k_total:
        kt = min(KWIN, k_total - k0)
        w_tile = w_ref[pl.ds(k0, kt), :]
        wn_tile = wn_ref[pl.ds(k0, kt), :]
        s2 = jax.lax.dot_general(
            w_tile, x, (((1,), (0,)), ((), ())),
            preferred_element_type=jnp.float32)
        mval = jnp.full((CHUNK, TM), jnp.inf, dtype=jnp.float32)
        midx = jnp.zeros((CHUNK, TM), dtype=jnp.int32)
        iota0 = jax.lax.broadcasted_iota(jnp.int32, (CHUNK, TM), 0)
        c0 = 0
        while c0 < kt:
            ch = min(CHUNK, kt - c0)
            d_c = (xn + wn_tile[c0:c0 + ch, :]) - s2[c0:c0 + ch, :]
            if ch < CHUNK:
                pad = jnp.full((CHUNK - ch, TM), jnp.inf, dtype=jnp.float32)
                d_c = jnp.concatenate([d_c, pad], axis=0)
            cond = d_c < mval
            mval = jnp.where(cond, d_c, mval)
            midx = jnp.where(cond, iota0 + (k0 + c0), midx)
            c0 += ch
        m = jnp.min(mval, axis=0, keepdims=True)
        idx = jnp.min(jnp.where(mval == m, midx, INT_MAX), axis=0, keepdims=True)
        take = m < r_val
        r_idx = jnp.where(take, idx, r_idx)
        r_val = jnp.where(take, m, r_val)
        r_val = r_val.astype(jnp.bfloat16).astype(jnp.float32)
        k0 += kt
    out_ref[...] = r_idx.reshape(1, 1, TM)


@functools.partial(jax.jit, static_argnames=("interpret",))
def kernel(inputs, weight, interpret=False):
    B, C, T = inputs.shape
    K = weight.shape[0]
    tt = T // TM

    wnorm = pl.pallas_call(
        _wnorm_kernel,
        out_shape=jax.ShapeDtypeStruct((K, 1), jnp.float32),
        interpret=interpret,
    )(weight)

    xnorm = pl.pallas_call(
        _xnorm_kernel,
        grid=(B * tt,),
        in_specs=[pl.BlockSpec((1, C, TM), lambda n, tt=tt: (n // tt, 0, n % tt))],
        out_specs=pl.BlockSpec((1, 1, TM), lambda n: (n, 0, 0)),
        out_shape=jax.ShapeDtypeStruct((B * tt, 1, TM), jnp.float32),
        interpret=interpret,
    )(inputs)

    x_bf = (inputs.astype(jnp.bfloat16) * 2)
    w_bf = weight.astype(jnp.bfloat16)

    out = pl.pallas_call(
        _vq_kernel,
        grid=(B * tt,),
        in_specs=[
            pl.BlockSpec((1, C, TM), lambda n, tt=tt: (n // tt, 0, n % tt)),
            pl.BlockSpec((1, 1, TM), lambda n: (n, 0, 0)),
            pl.BlockSpec((K, C), lambda n: (0, 0)),
            pl.BlockSpec((K, 1), lambda n: (0, 0)),
        ],
        out_specs=pl.BlockSpec((1, 1, TM), lambda n: (n, 0, 0)),
        out_shape=jax.ShapeDtypeStruct((B * tt, 1, TM), jnp.int32),
        interpret=interpret,
    )(x_bf, xnorm, w_bf, wnorm)

    return out.reshape(B, T)


# speedup vs baseline: 1.6412x; 1.2824x over previous
import functools

import jax
import jax.numpy as jnp
from jax.experimental import pallas as pl

TM = 512
KWIN = 2736
INT_MAX = 2**31 - 1


def _wnorm_kernel(w_ref, out_ref):
    w = w_ref[...]
    out_ref[...] = jnp.sum(w * w, axis=1, keepdims=True)


def _xnorm_kernel(x_ref, out_ref):
    x = x_ref[0]
    out_ref[...] = jnp.sum(x * x, axis=0, keepdims=True).reshape(1, 1, TM)


CHUNK = 48


def _vq_kernel(x_ref, xn_ref, w_ref, wn_ref, out_ref):
    x = x_ref[0]
    xn = xn_ref[0]
    k_total = w_ref.shape[0]
    r_val = jnp.full((1, TM), jnp.inf, dtype=jnp.float32)
    r_idx = jnp.zeros((1, TM), dtype=jnp.int32)
    k0 = 0
    while k0 < k_total:
        kt = min(KWIN, k_total - k0)
        w_tile = w_ref[pl.ds(k0, kt), :]
        wn_tile = wn_ref[pl.ds(k0, kt), :]
        s2 = jax.lax.dot_general(
            w_tile, x, (((1,), (0,)), ((), ())),
            preferred_element_type=jnp.float32)
        mval = jnp.full((CHUNK, TM), jnp.inf, dtype=jnp.float32)
        midx = jnp.zeros((CHUNK, TM), dtype=jnp.int32)
        iota0 = jax.lax.broadcasted_iota(jnp.int32, (CHUNK, TM), 0)
        c0 = 0
        while c0 < kt:
            ch = min(CHUNK, kt - c0)
            d_c = (xn + wn_tile[c0:c0 + ch, :]) - s2[c0:c0 + ch, :]
            if ch < CHUNK:
                pad = jnp.full((CHUNK - ch, TM), jnp.inf, dtype=jnp.float32)
                d_c = jnp.concatenate([d_c, pad], axis=0)
            cond = d_c < mval
            mval = jnp.where(cond, d_c, mval)
            midx = jnp.where(cond, iota0 + (k0 + c0), midx)
            c0 += ch
        m = jnp.min(mval, axis=0, keepdims=True)
        idx = jnp.min(jnp.where(mval == m, midx, INT_MAX), axis=0, keepdims=True)
        take = m < r_val
        r_idx = jnp.where(take, idx, r_idx)
        r_val = jnp.where(take, m, r_val)
        r_val = r_val.astype(jnp.bfloat16).astype(jnp.float32)
        k0 += kt
    out_ref[...] = r_idx.reshape(1, 1, TM)


@functools.partial(jax.jit, static_argnames=("interpret",))
def kernel(inputs, weight, interpret=False):
    B, C, T = inputs.shape
    K = weight.shape[0]
    tt = T // TM

    wnorm = pl.pallas_call(
        _wnorm_kernel,
        out_shape=jax.ShapeDtypeStruct((K, 1), jnp.float32),
        interpret=interpret,
    )(weight)

    xnorm = pl.pallas_call(
        _xnorm_kernel,
        grid=(B * tt,),
        in_specs=[pl.BlockSpec((1, C, TM), lambda n, tt=tt: (n // tt, 0, n % tt))],
        out_specs=pl.BlockSpec((1, 1, TM), lambda n: (n, 0, 0)),
        out_shape=jax.ShapeDtypeStruct((B * tt, 1, TM), jnp.float32),
        interpret=interpret,
    )(inputs)

    x_bf = (inputs.astype(jnp.bfloat16) * 2)
    w_bf = weight.astype(jnp.bfloat16)

    out = pl.pallas_call(
        _vq_kernel,
        grid=(B * tt,),
        in_specs=[
            pl.BlockSpec((1, C, TM), lambda n, tt=tt: (n // tt, 0, n % tt)),
            pl.BlockSpec((1, 1, TM), lambda n: (n, 0, 0)),
            pl.BlockSpec((K, C), lambda n: (0, 0)),
            pl.BlockSpec((K, 1), lambda n: (0, 0)),
        ],
        out_specs=pl.BlockSpec((1, 1, TM), lambda n: (n, 0, 0)),
        out_shape=jax.ShapeDtypeStruct((B * tt, 1, TM), jnp.int32),
        interpret=interpret,
    )(x_bf, xnorm, w_bf, wnorm)

    return out.reshape(B, T)


# speedup vs baseline: 1.8182x; 1.1078x over previous
import functools

import jax
import jax.numpy as jnp
from jax.experimental import pallas as pl

TM = 512
KWIN = 2736
INT_MAX = 2**31 - 1


def _wnorm_kernel(w_ref, out_ref):
    w = w_ref[...]
    out_ref[...] = jnp.sum(w * w, axis=1, keepdims=True)


def _xnorm_kernel(x_ref, out_ref):
    x = x_ref[0]
    out_ref[...] = jnp.sum(x * x, axis=0, keepdims=True).reshape(1, 1, TM)


CHUNK = 24


def _vq_kernel(x_ref, xn_ref, w_ref, wn_ref, out_ref):
    x = x_ref[0]
    xn = xn_ref[0]
    k_total = w_ref.shape[0]
    r_val = jnp.full((1, TM), jnp.inf, dtype=jnp.float32)
    r_idx = jnp.zeros((1, TM), dtype=jnp.int32)
    k0 = 0
    while k0 < k_total:
        kt = min(KWIN, k_total - k0)
        w_tile = w_ref[pl.ds(k0, kt), :]
        wn_tile = wn_ref[pl.ds(k0, kt), :]
        s2 = jax.lax.dot_general(
            w_tile, x, (((1,), (0,)), ((), ())),
            preferred_element_type=jnp.float32)
        mval = jnp.full((CHUNK, TM), jnp.inf, dtype=jnp.float32)
        midx = jnp.zeros((CHUNK, TM), dtype=jnp.int32)
        iota0 = jax.lax.broadcasted_iota(jnp.int32, (CHUNK, TM), 0)
        c0 = 0
        while c0 < kt:
            ch = min(CHUNK, kt - c0)
            d_c = (xn + wn_tile[c0:c0 + ch, :]) - s2[c0:c0 + ch, :]
            if ch < CHUNK:
                pad = jnp.full((CHUNK - ch, TM), jnp.inf, dtype=jnp.float32)
                d_c = jnp.concatenate([d_c, pad], axis=0)
            cond = d_c < mval
            mval = jnp.where(cond, d_c, mval)
            midx = jnp.where(cond, iota0 + (k0 + c0), midx)
            c0 += ch
        m = jnp.min(mval, axis=0, keepdims=True)
        idx = jnp.min(jnp.where(mval == m, midx, INT_MAX), axis=0, keepdims=True)
        take = m < r_val
        r_idx = jnp.where(take, idx, r_idx)
        r_val = jnp.where(take, m, r_val)
        r_val = r_val.astype(jnp.bfloat16).astype(jnp.float32)
        k0 += kt
    out_ref[...] = r_idx.reshape(1, 1, TM)


@functools.partial(jax.jit, static_argnames=("interpret",))
def kernel(inputs, weight, interpret=False):
    B, C, T = inputs.shape
    K = weight.shape[0]
    tt = T // TM

    wnorm = pl.pallas_call(
        _wnorm_kernel,
        out_shape=jax.ShapeDtypeStruct((K, 1), jnp.float32),
        interpret=interpret,
    )(weight)

    xnorm = pl.pallas_call(
        _xnorm_kernel,
        grid=(B * tt,),
        in_specs=[pl.BlockSpec((1, C, TM), lambda n, tt=tt: (n // tt, 0, n % tt))],
        out_specs=pl.BlockSpec((1, 1, TM), lambda n: (n, 0, 0)),
        out_shape=jax.ShapeDtypeStruct((B * tt, 1, TM), jnp.float32),
        interpret=interpret,
    )(inputs)

    x_bf = (inputs.astype(jnp.bfloat16) * 2)
    w_bf = weight.astype(jnp.bfloat16)

    out = pl.pallas_call(
        _vq_kernel,
        grid=(B * tt,),
        in_specs=[
            pl.BlockSpec((1, C, TM), lambda n, tt=tt: (n // tt, 0, n % tt)),
            pl.BlockSpec((1, 1, TM), lambda n: (n, 0, 0)),
            pl.BlockSpec((K, C), lambda n: (0, 0)),
            pl.BlockSpec((K, 1), lambda n: (0, 0)),
        ],
        out_specs=pl.BlockSpec((1, 1, TM), lambda n: (n, 0, 0)),
        out_shape=jax.ShapeDtypeStruct((B * tt, 1, TM), jnp.int32),
        interpret=interpret,
    )(x_bf, xnorm, w_bf, wnorm)

    return out.reshape(B, T)


# speedup vs baseline: 1.8359x; 1.0097x over previous
import functools

import jax
import jax.numpy as jnp
from jax.experimental import pallas as pl

TM = 512
KWIN = 2736
INT_MAX = 2**31 - 1


def _wnorm_kernel(w_ref, out_ref):
    w = w_ref[...]
    out_ref[...] = jnp.sum(w * w, axis=1, keepdims=True)


def _xnorm_kernel(x_ref, out_ref):
    x = x_ref[0]
    out_ref[...] = jnp.sum(x * x, axis=0, keepdims=True).reshape(1, 1, TM)


CHUNK = 16


def _vq_kernel(x_ref, xn_ref, w_ref, wn_ref, out_ref):
    x = x_ref[0]
    xn = xn_ref[0]
    k_total = w_ref.shape[0]
    r_val = jnp.full((1, TM), jnp.inf, dtype=jnp.float32)
    r_idx = jnp.zeros((1, TM), dtype=jnp.int32)
    k0 = 0
    while k0 < k_total:
        kt = min(KWIN, k_total - k0)
        w_tile = w_ref[pl.ds(k0, kt), :]
        wn_tile = wn_ref[pl.ds(k0, kt), :]
        s2 = jax.lax.dot_general(
            w_tile, x, (((1,), (0,)), ((), ())),
            preferred_element_type=jnp.float32)
        mval = jnp.full((CHUNK, TM), jnp.inf, dtype=jnp.float32)
        midx = jnp.zeros((CHUNK, TM), dtype=jnp.int32)
        iota0 = jax.lax.broadcasted_iota(jnp.int32, (CHUNK, TM), 0)
        c0 = 0
        while c0 < kt:
            ch = min(CHUNK, kt - c0)
            d_c = (xn + wn_tile[c0:c0 + ch, :]) - s2[c0:c0 + ch, :]
            if ch < CHUNK:
                pad = jnp.full((CHUNK - ch, TM), jnp.inf, dtype=jnp.float32)
                d_c = jnp.concatenate([d_c, pad], axis=0)
            cond = d_c < mval
            mval = jnp.where(cond, d_c, mval)
            midx = jnp.where(cond, iota0 + (k0 + c0), midx)
            c0 += ch
        m = jnp.min(mval, axis=0, keepdims=True)
        idx = jnp.min(jnp.where(mval == m, midx, INT_MAX), axis=0, keepdims=True)
        take = m < r_val
        r_idx = jnp.where(take, idx, r_idx)
        r_val = jnp.where(take, m, r_val)
        r_val = r_val.astype(jnp.bfloat16).astype(jnp.float32)
        k0 += kt
    out_ref[...] = r_idx.reshape(1, 1, TM)


@functools.partial(jax.jit, static_argnames=("interpret",))
def kernel(inputs, weight, interpret=False):
    B, C, T = inputs.shape
    K = weight.shape[0]
    tt = T // TM

    wnorm = pl.pallas_call(
        _wnorm_kernel,
        out_shape=jax.ShapeDtypeStruct((K, 1), jnp.float32),
        interpret=interpret,
    )(weight)

    xnorm = pl.pallas_call(
        _xnorm_kernel,
        grid=(B * tt,),
        in_specs=[pl.BlockSpec((1, C, TM), lambda n, tt=tt: (n // tt, 0, n % tt))],
        out_specs=pl.BlockSpec((1, 1, TM), lambda n: (n, 0, 0)),
        out_shape=jax.ShapeDtypeStruct((B * tt, 1, TM), jnp.float32),
        interpret=interpret,
    )(inputs)

    x_bf = (inputs.astype(jnp.bfloat16) * 2)
    w_bf = weight.astype(jnp.bfloat16)

    out = pl.pallas_call(
        _vq_kernel,
        grid=(B * tt,),
        in_specs=[
            pl.BlockSpec((1, C, TM), lambda n, tt=tt: (n // tt, 0, n % tt)),
            pl.BlockSpec((1, 1, TM), lambda n: (n, 0, 0)),
            pl.BlockSpec((K, C), lambda n: (0, 0)),
            pl.BlockSpec((K, 1), lambda n: (0, 0)),
        ],
        out_specs=pl.BlockSpec((1, 1, TM), lambda n: (n, 0, 0)),
        out_shape=jax.ShapeDtypeStruct((B * tt, 1, TM), jnp.int32),
        interpret=interpret,
    )(x_bf, xnorm, w_bf, wnorm)

    return out.reshape(B, T)
